# baseline (speedup 1.0000x reference)
import functools

import jax
import jax.numpy as jnp
from jax import lax
from jax.experimental import pallas as pl
from jax.experimental.pallas import tpu as pltpu
from jax.experimental.pallas import tpu_sc as plsc

_NC = 2
_NS = 16
_NW = _NC * _NS

_BT = 128
_NG = 4
_NT = 2


@functools.lru_cache(maxsize=None)
def _make_gather(batch, hist, d_model):
    assert batch % (_NW * _BT) == 0
    assert d_model == 64
    blocks_per_w = batch // (_NW * _BT)
    n_dt = d_model // 8

    mesh = plsc.VectorSubcoreMesh(core_axis_name="c", subcore_axis_name="s")

    @functools.partial(
        pl.kernel,
        mesh=mesh,
        out_type=jax.ShapeDtypeStruct(
            (hist, n_dt, batch // _BT, 8, _BT), jnp.float32
        ),
        compiler_params=pltpu.CompilerParams(
            use_tc_tiling_on_sc=False, needs_layout_passes=False
        ),
        scratch_types=[
            pltpu.VMEM((_BT, hist), jnp.int32),
            pltpu.VMEM((hist, _BT), jnp.int32),
            pltpu.VMEM((_NG, _BT, d_model), jnp.float32),
            pltpu.VMEM((_NT, n_dt, 8, _BT), jnp.float32),
            pltpu.SemaphoreType.DMA,
            pltpu.SemaphoreType.DMA,
            pltpu.SemaphoreType.DMA,
            pltpu.SemaphoreType.DMA,
            pltpu.SemaphoreType.DMA,
            pltpu.SemaphoreType.DMA,
        ],
    )
    def gather_kernel(x_hbm, table_hbm, out_hbm, idx_v, idxt_v, g_v, gt_v,
                      gs0, gs1, gs2, gs3, os0, os1):
        gsem = [gs0, gs1, gs2, gs3]
        osem = [os0, os1]
        wid = lax.axis_index("s") * _NC + lax.axis_index("c")
        iota16 = lax.iota(jnp.int32, 16)
        bvecs = [iota16 + c * 16 for c in range(_BT // 16)]

        def gather_copy(h, gb):
            return pltpu.make_async_copy(
                table_hbm.at[idxt_v.at[h]], g_v.at[gb], gsem[gb]
            )

        def store_copy(h, bt, tb):
            return pltpu.make_async_copy(
                gt_v.at[tb], out_hbm.at[h, pl.ds(0, n_dt), bt], osem[tb]
            )

        def transpose_unit(gb, tb):
            def dt_body(dt, c):
                for di in range(8):
                    dvec = jnp.full((16,), dt * 8 + di, jnp.int32)
                    for c8 in range(_BT // 16):
                        v = plsc.load_gather(g_v.at[gb], [bvecs[c8], dvec])
                        gt_v[tb, dt, di, pl.ds(c8 * 16, 16)] = v
                return c

            lax.fori_loop(0, n_dt, dt_body, 0)

        def block(blk, carry):
            bt = wid * blocks_per_w + blk

            pltpu.sync_copy(x_hbm.at[pl.ds(bt * _BT, _BT)], idx_v)

            def idxt_body(h, c):
                hvec = jnp.full((16,), h, jnp.int32)
                for c8 in range(_BT // 16):
                    v = plsc.load_gather(idx_v, [bvecs[c8], hvec])
                    idxt_v[h, pl.ds(c8 * 16, 16)] = v
                return c

            lax.fori_loop(0, hist, idxt_body, 0)

            for lane in range(_NG):
                gather_copy(lane, lane).start()

            n_super = (hist + _NG - 1) // _NG

            def super_iter(t, c):
                for lane in range(_NG):
                    h = t * _NG + lane

                    @pl.when(h < hist)
                    def _():
                        gather_copy(h, lane).wait()

                        @pl.when(h >= _NT)
                        def _():
                            store_copy(h - _NT, bt, lane % _NT).wait()

                        transpose_unit(lane, lane % _NT)

                        @pl.when(h + _NG < hist)
                        def _():
                            gather_copy(h + _NG, lane).start()

                        store_copy(h, bt, lane % _NT).start()

                return c

            lax.fori_loop(0, n_super, super_iter, 0)

            for tail in range(_NT):
                store_copy(hist - _NT + tail, bt, (hist - _NT + tail) % _NT).wait()
            return carry

        lax.fori_loop(0, blocks_per_w, block, 0)

    return gather_kernel


def kernel(x, table):
    batch, hist = x.shape
    d_model = table.shape[1]
    out5 = _make_gather(batch, hist, d_model)(x.astype(jnp.int32), table)
    return jnp.transpose(out5, (2, 4, 0, 1, 3)).reshape(batch, hist, d_model)


# speedup vs baseline: 1.2843x; 1.2843x over previous
import functools

import jax
import jax.numpy as jnp
from jax import lax
from jax.experimental import pallas as pl
from jax.experimental.pallas import tpu as pltpu
from jax.experimental.pallas import tpu_sc as plsc

_NC = 2
_NS = 16
_NW = _NC * _NS

_BT = 128
_NG = 4
_NT = 2


@functools.lru_cache(maxsize=None)
def _make_gather(batch, hist, d_model):
    assert batch % (_NW * _BT) == 0
    assert d_model == 64
    blocks_per_w = batch // (_NW * _BT)
    n_dt = d_model // 8

    mesh = plsc.VectorSubcoreMesh(core_axis_name="c", subcore_axis_name="s")

    @functools.partial(
        pl.kernel,
        mesh=mesh,
        out_type=jax.ShapeDtypeStruct(
            (hist, n_dt, batch // _BT, 8, _BT), jnp.float32
        ),
        compiler_params=pltpu.CompilerParams(
            use_tc_tiling_on_sc=False, needs_layout_passes=False
        ),
        scratch_types=[
            pltpu.VMEM((_BT, hist), jnp.int32),
            pltpu.VMEM((hist, _BT), jnp.int32),
            pltpu.VMEM((_NG, _BT, d_model), jnp.float32),
            pltpu.VMEM((_NT, n_dt, 8, _BT), jnp.float32),
            pltpu.SemaphoreType.DMA,
            pltpu.SemaphoreType.DMA,
            pltpu.SemaphoreType.DMA,
            pltpu.SemaphoreType.DMA,
            pltpu.SemaphoreType.DMA,
            pltpu.SemaphoreType.DMA,
        ],
    )
    def gather_kernel(x_hbm, table_hbm, out_hbm, idx_v, idxt_v, g_v, gt_v,
                      gs0, gs1, gs2, gs3, os0, os1):
        gsem = [gs0, gs1, gs2, gs3]
        osem = [os0, os1]
        wid = lax.axis_index("s") * _NC + lax.axis_index("c")
        iota16 = lax.iota(jnp.int32, 16)
        bvecs = [iota16 + c * 16 for c in range(_BT // 16)]

        def gather_copy(h, gb):
            return pltpu.make_async_copy(
                table_hbm.at[idxt_v.at[h]], g_v.at[gb], gsem[gb]
            )

        def store_copy(h, bt, tb):
            return pltpu.make_async_copy(
                gt_v.at[tb], out_hbm.at[h, pl.ds(0, n_dt), bt], osem[tb]
            )

        def transpose_unit(gb, tb):
            @plsc.parallel_loop(0, n_dt, unroll=2)
            def dt_body(dt):
                for di in range(8):
                    dvec = jnp.full((16,), dt * 8 + di, jnp.int32)
                    for c8 in range(_BT // 16):
                        v = plsc.load_gather(g_v.at[gb], [bvecs[c8], dvec])
                        gt_v[tb, dt, di, pl.ds(c8 * 16, 16)] = v

        def block(blk, carry):
            bt = wid * blocks_per_w + blk

            pltpu.sync_copy(x_hbm.at[pl.ds(bt * _BT, _BT)], idx_v)

            @plsc.parallel_loop(0, hist, unroll=2)
            def idxt_body(h):
                hvec = jnp.full((16,), h, jnp.int32)
                for c8 in range(_BT // 16):
                    v = plsc.load_gather(idx_v, [bvecs[c8], hvec])
                    idxt_v[h, pl.ds(c8 * 16, 16)] = v

            for lane in range(_NG):
                gather_copy(lane, lane).start()

            n_super = (hist + _NG - 1) // _NG

            def super_iter(t, c):
                for lane in range(_NG):
                    h = t * _NG + lane

                    @pl.when(h < hist)
                    def _():
                        gather_copy(h, lane).wait()

                        @pl.when(h >= _NT)
                        def _():
                            store_copy(h - _NT, bt, lane % _NT).wait()

                        transpose_unit(lane, lane % _NT)

                        @pl.when(h + _NG < hist)
                        def _():
                            gather_copy(h + _NG, lane).start()

                        store_copy(h, bt, lane % _NT).start()

                return c

            lax.fori_loop(0, n_super, super_iter, 0)

            for tail in range(_NT):
                store_copy(hist - _NT + tail, bt, (hist - _NT + tail) % _NT).wait()
            return carry

        lax.fori_loop(0, blocks_per_w, block, 0)

    return gather_kernel


def kernel(x, table):
    batch, hist = x.shape
    d_model = table.shape[1]
    out5 = _make_gather(batch, hist, d_model)(x.astype(jnp.int32), table)
    return jnp.transpose(out5, (2, 4, 0, 1, 3)).reshape(batch, hist, d_model)


# speedup vs baseline: 1.4299x; 1.1134x over previous
import functools

import jax
import jax.numpy as jnp
from jax import lax
from jax.experimental import pallas as pl
from jax.experimental.pallas import tpu as pltpu
from jax.experimental.pallas import tpu_sc as plsc

_NC = 2
_NS = 16
_NW = _NC * _NS

_BT = 128
_NG = 4
_NT = 2


@functools.lru_cache(maxsize=None)
def _make_gather(batch, hist, d_model):
    assert batch % (_NW * _BT) == 0
    assert d_model == 64
    blocks_per_w = batch // (_NW * _BT)
    n_dt = d_model // 8

    mesh = plsc.VectorSubcoreMesh(core_axis_name="c", subcore_axis_name="s")

    @functools.partial(
        pl.kernel,
        mesh=mesh,
        out_type=jax.ShapeDtypeStruct(
            (hist, n_dt, batch // _BT, 8, _BT), jnp.float32
        ),
        compiler_params=pltpu.CompilerParams(
            use_tc_tiling_on_sc=False, needs_layout_passes=False
        ),
        scratch_types=[
            pltpu.VMEM((_BT, hist), jnp.int32),
            pltpu.VMEM((hist, _BT), jnp.int32),
            pltpu.VMEM((_NG, _BT, d_model), jnp.float32),
            pltpu.VMEM((_NT, n_dt, 8, _BT), jnp.float32),
            pltpu.SemaphoreType.DMA,
            pltpu.SemaphoreType.DMA,
            pltpu.SemaphoreType.DMA,
            pltpu.SemaphoreType.DMA,
            pltpu.SemaphoreType.DMA,
            pltpu.SemaphoreType.DMA,
        ],
    )
    def gather_kernel(x_hbm, table_hbm, out_hbm, idx_v, idxt_v, g_v, gt_v,
                      gs0, gs1, gs2, gs3, os0, os1):
        gsem = [gs0, gs1, gs2, gs3]
        osem = [os0, os1]
        wid = lax.axis_index("s") * _NC + lax.axis_index("c")
        iota16 = lax.iota(jnp.int32, 16)
        bvecs = [iota16 + c * 16 for c in range(_BT // 16)]

        def gather_copy(h, gb):
            return pltpu.make_async_copy(
                table_hbm.at[idxt_v.at[h]], g_v.at[gb], gsem[gb]
            )

        def store_copy(h, bt, tb):
            return pltpu.make_async_copy(
                gt_v.at[tb], out_hbm.at[h, pl.ds(0, n_dt), bt], osem[tb]
            )

        def transpose_unit(gb, tb):
            @plsc.parallel_loop(0, n_dt, unroll=4)
            def dt_body(dt):
                for di in range(8):
                    dvec = jnp.full((16,), dt * 8 + di, jnp.int32)
                    for c8 in range(_BT // 16):
                        v = plsc.load_gather(g_v.at[gb], [bvecs[c8], dvec])
                        gt_v[tb, dt, di, pl.ds(c8 * 16, 16)] = v

        def block(blk, carry):
            bt = wid * blocks_per_w + blk

            pltpu.sync_copy(x_hbm.at[pl.ds(bt * _BT, _BT)], idx_v)

            @plsc.parallel_loop(0, hist, unroll=2)
            def idxt_body(h):
                hvec = jnp.full((16,), h, jnp.int32)
                for c8 in range(_BT // 16):
                    v = plsc.load_gather(idx_v, [bvecs[c8], hvec])
                    idxt_v[h, pl.ds(c8 * 16, 16)] = v

            for lane in range(_NG):
                gather_copy(lane, lane).start()

            n_super = (hist + _NG - 1) // _NG

            def super_iter(t, c):
                for lane in range(_NG):
                    h = t * _NG + lane

                    @pl.when(h < hist)
                    def _():
                        gather_copy(h, lane).wait()

                        @pl.when(h >= _NT)
                        def _():
                            store_copy(h - _NT, bt, lane % _NT).wait()

                        transpose_unit(lane, lane % _NT)

                        @pl.when(h + _NG < hist)
                        def _():
                            gather_copy(h + _NG, lane).start()

                        store_copy(h, bt, lane % _NT).start()

                return c

            lax.fori_loop(0, n_super, super_iter, 0)

            for tail in range(_NT):
                store_copy(hist - _NT + tail, bt, (hist - _NT + tail) % _NT).wait()
            return carry

        lax.fori_loop(0, blocks_per_w, block, 0)

    return gather_kernel


def kernel(x, table):
    batch, hist = x.shape
    d_model = table.shape[1]
    out5 = _make_gather(batch, hist, d_model)(x.astype(jnp.int32), table)
    return jnp.transpose(out5, (2, 4, 0, 1, 3)).reshape(batch, hist, d_model)


# speedup vs baseline: 2.0070x; 1.4036x over previous
import functools

import jax
import jax.numpy as jnp
from jax import lax
from jax.experimental import pallas as pl
from jax.experimental.pallas import tpu as pltpu
from jax.experimental.pallas import tpu_sc as plsc

_NC = 2
_NS = 16
_NW = _NC * _NS

_PADD = 128
_KB = 8
_NBUF = 2


def _pad8(n):
    return (n + 7) // 8 * 8


@functools.lru_cache(maxsize=None)
def _make_gather(batch, hist, d_model):
    assert batch % (_NW * _KB * _NBUF) == 0
    b_per_w = batch // _NW
    n_super = b_per_w // (_KB * _NBUF)
    hist_p = _pad8(hist)

    mesh = plsc.VectorSubcoreMesh(core_axis_name="c", subcore_axis_name="s")

    @functools.partial(
        pl.kernel,
        mesh=mesh,
        out_type=jax.ShapeDtypeStruct((batch * hist_p, _PADD), jnp.float32),
        compiler_params=pltpu.CompilerParams(use_tc_tiling_on_sc=False),
        scratch_types=[
            pltpu.VMEM((_NBUF, _KB, hist), jnp.int32),
            pltpu.VMEM((_NBUF, _KB * hist, d_model), jnp.float32),
            pltpu.SemaphoreType.DMA,
            pltpu.SemaphoreType.DMA,
            pltpu.SemaphoreType.DMA,
            pltpu.SemaphoreType.DMA,
        ],
    )
    def gather_kernel(x_hbm, table_hbm, out_hbm, idx_v, rows_v, g0, g1, o0, o1):
        gsem = [g0, g1]
        osem = [o0, o1]
        wid = lax.axis_index("s") * _NC + lax.axis_index("c")
        b_base = wid * b_per_w

        def out_src_dst(b, b0):
            for j in range(_KB):
                yield (
                    rows_v.at[b].at[pl.ds(j * hist, hist)],
                    out_hbm.at[pl.ds((b0 + j) * hist_p, hist), pl.ds(0, d_model)],
                )

        def super_iter(t, carry):
            for b in range(_NBUF):
                b0 = b_base + (t * _NBUF + b) * _KB

                @pl.when(t > 0)
                def _():
                    for src, dst in out_src_dst(b, b0):
                        pltpu.make_async_copy(src, dst, osem[b]).wait()

                pltpu.sync_copy(x_hbm.at[pl.ds(b0, _KB)], idx_v.at[b])
                for j in range(_KB):
                    pltpu.async_copy(
                        table_hbm.at[idx_v.at[b].at[j]],
                        rows_v.at[b].at[pl.ds(j * hist, hist)],
                        gsem[b],
                    )

            for b in range(_NBUF):
                b0 = b_base + (t * _NBUF + b) * _KB
                for j in range(_KB):
                    pltpu.make_async_copy(
                        table_hbm.at[idx_v.at[b].at[j]],
                        rows_v.at[b].at[pl.ds(j * hist, hist)],
                        gsem[b],
                    ).wait()
                for src, dst in out_src_dst(b, b0):
                    pltpu.async_copy(src, dst, osem[b])
            return carry

        lax.fori_loop(0, n_super, super_iter, 0)

        for b in range(_NBUF):
            b0 = b_base + ((n_super - 1) * _NBUF + b) * _KB
            for src, dst in out_src_dst(b, b0):
                pltpu.make_async_copy(src, dst, osem[b]).wait()

    return gather_kernel


def kernel(x, table):
    batch, hist = x.shape
    d_model = table.shape[1]
    hist_p = _pad8(hist)
    out_p = _make_gather(batch, hist, d_model)(x.astype(jnp.int32), table)
    return out_p.reshape(batch, hist_p, _PADD)[:, :hist, :d_model]
